# baseline (speedup 1.0000x reference)
import functools

import jax
import jax.numpy as jnp
from jax import lax
from jax.experimental import pallas as pl
from jax.experimental.pallas import tpu as pltpu
from jax.experimental.pallas import tpu_sc as plsc

_B = 16384
_K = 1024
_D = 256
_H = 512
_P = 784

_BS = 1024


def _encoder_body(x_ref, w1_ref, b1_ref, w2_ref, b2_ref, cb_ref,
                  ze_ref, inds_ref):
    xb = x_ref[...].reshape(_BS, _P)
    h = jnp.maximum(
        jnp.dot(xb, w1_ref[...], preferred_element_type=jnp.float32)
        + b1_ref[...], 0.0)
    z_e = (jnp.dot(h, w2_ref[...], preferred_element_type=jnp.float32)
           + b2_ref[...])
    ze_ref[...] = z_e
    cb = cb_ref[...]
    d2 = (jnp.sum(z_e * z_e, axis=1, keepdims=True)
          + jnp.sum(cb * cb, axis=1)[None, :]
          - 2.0 * jnp.dot(z_e, cb.T, preferred_element_type=jnp.float32))
    m = jnp.min(d2, axis=1, keepdims=True)
    iota = lax.broadcasted_iota(jnp.int32, d2.shape, 1)
    inds_ref[...] = jnp.min(jnp.where(d2 <= m, iota, _K), axis=1)


def _decoder_body(inds_ref, cb_ref, w3_ref, b3_ref, w4_ref, b4_ref, out_ref):
    onehot = jnp.where(
        lax.broadcasted_iota(jnp.int32, (_BS, _K), 1) == inds_ref[...][:, None],
        1.0, 0.0)
    z_q = jnp.dot(onehot, cb_ref[...], preferred_element_type=jnp.float32)
    g = jnp.maximum(
        jnp.dot(z_q, w3_ref[...], preferred_element_type=jnp.float32)
        + b3_ref[...], 0.0)
    y = jax.nn.sigmoid(
        jnp.dot(g, w4_ref[...], preferred_element_type=jnp.float32)
        + b4_ref[...])
    out_ref[...] = y.reshape(_BS, 1, 28, 28)


def _full(shape):
    return pl.BlockSpec(shape, lambda i: (0,) * len(shape))


def _encoder(x4, W1, b1, W2, b2, codebook):
    grid = _B // _BS
    return pl.pallas_call(
        _encoder_body,
        grid=(grid,),
        in_specs=[
            pl.BlockSpec((_BS, 1, 28, 28), lambda i: (i, 0, 0, 0)),
            _full((_P, _H)), _full((_H,)),
            _full((_H, _D)), _full((_D,)),
            _full((_K, _D)),
        ],
        out_specs=[
            pl.BlockSpec((_BS, _D), lambda i: (i, 0)),
            pl.BlockSpec((_BS,), lambda i: (i,)),
        ],
        out_shape=[
            jax.ShapeDtypeStruct((_B, _D), jnp.float32),
            jax.ShapeDtypeStruct((_B,), jnp.int32),
        ],
        compiler_params=pltpu.CompilerParams(
            dimension_semantics=("arbitrary",)),
    )(x4, W1, b1, W2, b2, codebook)


def _decoder(inds, codebook, W3, b3, W4, b4):
    grid = _B // _BS
    return pl.pallas_call(
        _decoder_body,
        grid=(grid,),
        in_specs=[
            pl.BlockSpec((_BS,), lambda i: (i,)),
            _full((_K, _D)),
            _full((_D, _H)), _full((_H,)),
            _full((_H, _P)), _full((_P,)),
        ],
        out_specs=pl.BlockSpec((_BS, 1, 28, 28), lambda i: (i, 0, 0, 0)),
        out_shape=jax.ShapeDtypeStruct((_B, 1, 28, 28), jnp.float32),
        compiler_params=pltpu.CompilerParams(
            dimension_semantics=("arbitrary",)),
    )(inds, codebook, W3, b3, W4, b4)


_SC_INFO = plsc.get_sparse_core_info()
_NC = _SC_INFO.num_cores
_NS = _SC_INFO.num_subcores
_NW = _NC * _NS
_BPW = _B // _NW
_CHUNK = 128
_NBUF = 3
_NCH = _BPW // _CHUNK
_CBS = _K // _NS


@functools.partial(
    pl.kernel,
    mesh=plsc.VectorSubcoreMesh(core_axis_name="c", subcore_axis_name="s"),
    out_type=jax.ShapeDtypeStruct((_B, _D), jnp.float32),
    scratch_types=[
        pltpu.VMEM((_BPW,), jnp.int32),
        pltpu.VMEM((2, _CHUNK, _D), jnp.float32),
        pltpu.SemaphoreType.DMA,
        pltpu.SemaphoreType.DMA,
        pltpu.SemaphoreType.DMA,
        pltpu.SemaphoreType.DMA,
        pltpu.SemaphoreType.DMA,
    ],
)
def _sc_gather(cb_hbm, idx_hbm, out_hbm, idx_v, rows_v,
               isem, gsem0, gsem1, osem0, osem1):
    sid = lax.axis_index("s")
    wid = sid * _NC + lax.axis_index("c")
    base = wid * _BPW
    pltpu.async_copy(idx_hbm.at[pl.ds(base, _BPW)], idx_v, isem).wait()

    gsem = (gsem0, gsem1)
    osem = (osem0, osem1)

    def _gather(c):
        return pltpu.async_copy(
            cb_hbm.at[idx_v.at[pl.ds(c * _CHUNK, _CHUNK)]],
            rows_v.at[c % 2], gsem[c % 2])

    def _writeback(c):
        return pltpu.async_copy(
            rows_v.at[c % 2], out_hbm.at[pl.ds(base + c * _CHUNK, _CHUNK)],
            osem[c % 2])

    g0 = _gather(0)
    g1 = _gather(1)
    g0.wait()
    o0 = _writeback(0)
    g1.wait()
    o1 = _writeback(1)
    o0.wait()
    g2 = _gather(2)
    o1.wait()
    g3 = _gather(3)
    g2.wait()
    o2 = _writeback(2)
    g3.wait()
    o3 = _writeback(3)
    o2.wait()
    o3.wait()


def kernel(x, W1, b1, W2, b2, codebook, W3, b3, W4, b4):
    z_e, inds = _encoder(x, W1, b1, W2, b2, codebook)
    z_q = _sc_gather(codebook, inds)
    x_hat = _decoder(inds, codebook, W3, b3, W4, b4)
    return x_hat, z_e, z_q, inds


# speedup vs baseline: 3.0340x; 3.0340x over previous
import functools

import jax
import jax.numpy as jnp
from jax import lax
from jax.experimental import pallas as pl
from jax.experimental.pallas import tpu as pltpu
from jax.experimental.pallas import tpu_sc as plsc

_B = 16384
_K = 1024
_D = 256
_H = 512
_P = 784

_BS = 1024
_BT = _BS // 128


def _dotT(w, x):
    return lax.dot_general(w, x, (((0,), (0,)), ((), ())),
                           preferred_element_type=jnp.float32)


def _encoder_body(x_ref, w1_ref, b1_ref, w2_ref, b2_ref, cb_ref,
                  ze_ref, inds_ref):
    xt = x_ref[...].reshape(_P, _BS)
    h = jnp.maximum(_dotT(w1_ref[...], xt) + b1_ref[...][:, None], 0.0)
    z_e = _dotT(w2_ref[...], h) + b2_ref[...][:, None]
    ze_ref[...] = z_e.T
    cb = cb_ref[...]
    d2 = (jnp.sum(z_e * z_e, axis=0)[None, :]
          + jnp.sum(cb * cb, axis=1)[:, None]
          - 2.0 * lax.dot_general(cb, z_e, (((1,), (0,)), ((), ())),
                                  preferred_element_type=jnp.float32))
    m = jnp.min(d2, axis=0)[None, :]
    iota = lax.broadcasted_iota(jnp.int32, d2.shape, 0)
    inds_ref[...] = jnp.min(jnp.where(d2 <= m, iota, _K), axis=0)


def _decoder_body(inds_ref, cb_ref, w3_ref, b3_ref, w4t_ref, b4_ref, out_ref):
    onehot = jnp.where(
        lax.broadcasted_iota(jnp.int32, (_K, _BS), 0) == inds_ref[...][None, :],
        1.0, 0.0)
    z_q = _dotT(cb_ref[...], onehot)
    g = jnp.maximum(_dotT(w3_ref[...], z_q) + b3_ref[...][:, None], 0.0)
    y = jax.nn.sigmoid(
        jnp.dot(w4t_ref[...], g, preferred_element_type=jnp.float32)
        + b4_ref[...][:, None])
    out_ref[...] = y.reshape(28, 28, _BT, 128)


def _full(shape):
    return pl.BlockSpec(shape, lambda i: (0,) * len(shape))


def _encoder(xt3, W1, b1, W2, b2, codebook):
    grid = _B // _BS
    return pl.pallas_call(
        _encoder_body,
        grid=(grid,),
        in_specs=[
            pl.BlockSpec((98, 8, _BT, 128), lambda i: (0, 0, i, 0)),
            _full((_P, _H)), _full((_H,)),
            _full((_H, _D)), _full((_D,)),
            _full((_K, _D)),
        ],
        out_specs=[
            pl.BlockSpec((_BS, _D), lambda i: (i, 0)),
            pl.BlockSpec((_BS,), lambda i: (i,)),
        ],
        out_shape=[
            jax.ShapeDtypeStruct((_B, _D), jnp.float32),
            jax.ShapeDtypeStruct((_B,), jnp.int32),
        ],
        compiler_params=pltpu.CompilerParams(
            dimension_semantics=("arbitrary",)),
    )(xt3, W1, b1, W2, b2, codebook)


def _decoder(inds, codebook, W3, b3, W4t, b4):
    grid = _B // _BS
    return pl.pallas_call(
        _decoder_body,
        grid=(grid,),
        in_specs=[
            pl.BlockSpec((_BS,), lambda i: (i,)),
            _full((_K, _D)),
            _full((_D, _H)), _full((_H,)),
            _full((_P, _H)), _full((_P,)),
        ],
        out_specs=pl.BlockSpec((28, 28, _BT, 128), lambda i: (0, 0, i, 0)),
        out_shape=jax.ShapeDtypeStruct((28, 28, 128, 128), jnp.float32),
        compiler_params=pltpu.CompilerParams(
            dimension_semantics=("arbitrary",)),
    )(inds, codebook, W3, b3, W4t, b4)


_SC_INFO = plsc.get_sparse_core_info()
_NC = _SC_INFO.num_cores
_NS = _SC_INFO.num_subcores
_NW = _NC * _NS
_BPW = _B // _NW
_CHUNK = 128
_NCH = _BPW // _CHUNK


@functools.partial(
    pl.kernel,
    mesh=plsc.VectorSubcoreMesh(core_axis_name="c", subcore_axis_name="s"),
    out_type=jax.ShapeDtypeStruct((_B, _D), jnp.float32),
    scratch_types=[
        pltpu.VMEM((_BPW,), jnp.int32),
        pltpu.VMEM((2, _CHUNK, _D), jnp.float32),
        pltpu.SemaphoreType.DMA,
        pltpu.SemaphoreType.DMA,
        pltpu.SemaphoreType.DMA,
        pltpu.SemaphoreType.DMA,
        pltpu.SemaphoreType.DMA,
    ],
)
def _sc_gather(cb_hbm, idx_hbm, out_hbm, idx_v, rows_v,
               isem, gsem0, gsem1, osem0, osem1):
    sid = lax.axis_index("s")
    wid = sid * _NC + lax.axis_index("c")
    base = wid * _BPW
    pltpu.async_copy(idx_hbm.at[pl.ds(base, _BPW)], idx_v, isem).wait()

    gsem = (gsem0, gsem1)
    osem = (osem0, osem1)

    def _gather(c):
        return pltpu.async_copy(
            cb_hbm.at[idx_v.at[pl.ds(c * _CHUNK, _CHUNK)]],
            rows_v.at[c % 2], gsem[c % 2])

    def _writeback(c):
        return pltpu.async_copy(
            rows_v.at[c % 2], out_hbm.at[pl.ds(base + c * _CHUNK, _CHUNK)],
            osem[c % 2])

    g0 = _gather(0)
    g1 = _gather(1)
    g0.wait()
    o0 = _writeback(0)
    g1.wait()
    o1 = _writeback(1)
    o0.wait()
    g2 = _gather(2)
    o1.wait()
    g3 = _gather(3)
    g2.wait()
    o2 = _writeback(2)
    g3.wait()
    o3 = _writeback(3)
    o2.wait()
    o3.wait()


def kernel(x, W1, b1, W2, b2, codebook, W3, b3, W4, b4):
    xt4 = x.transpose(2, 3, 1, 0).reshape(98, 8, 128, 128)
    z_e, inds = _encoder(xt4, W1, b1, W2, b2, codebook)
    z_q = _sc_gather(codebook, inds)
    y4 = _decoder(inds, codebook, W3, b3, W4.T, b4)
    x_hat = y4.transpose(2, 3, 0, 1).reshape(_B, 1, 28, 28)
    return x_hat, z_e, z_q, inds
